# baseline (speedup 1.0000x reference)
import functools

import jax
import jax.numpy as jnp
from jax import lax
from jax.experimental import pallas as pl
from jax.experimental.pallas import tpu as pltpu
from jax.experimental.pallas import tpu_sc as plsc

NBUF = 5
PREFETCH = 4


@functools.lru_cache(maxsize=None)
def _build(n_b: int, n_s: int, vocab: int, dim: int):
    info = plsc.get_sparse_core_info()
    nw = info.num_cores * info.num_subcores
    b_per_w = n_b // nw
    nstep = n_s
    assert n_b % nw == 0 and b_per_w <= 128 and nstep % NBUF == 0

    mesh = plsc.VectorSubcoreMesh(core_axis_name="c", subcore_axis_name="s")

    @functools.partial(
        pl.kernel,
        mesh=mesh,
        out_type=jax.ShapeDtypeStruct((n_s, n_b, dim), jnp.float32),
        scratch_types=(
            [pltpu.VMEM((nstep, b_per_w), jnp.int32)]
            + [pltpu.VMEM((b_per_w, dim), jnp.float32) for _ in range(NBUF)]
            + [pltpu.SemaphoreType.DMA for _ in range(2 * NBUF + 1)]
        ),
    )
    def gather_kernel(x_hbm, w_hbm, out_hbm, idx_v, *rest):
        bufs = rest[:NBUF]
        gsems = rest[NBUF : 2 * NBUF]
        ssems = rest[2 * NBUF : 3 * NBUF]
        xsem = rest[3 * NBUF]
        wid = lax.axis_index("s") * info.num_cores + lax.axis_index("c")
        b0 = wid * b_per_w

        def start_gather(t, b):
            pltpu.async_copy(w_hbm.at[idx_v.at[t]], bufs[b], gsems[b])

        def wait_gather(t, b):
            pltpu.make_async_copy(w_hbm.at[idx_v.at[t]], bufs[b], gsems[b]).wait()

        def start_store(t, b):
            pltpu.async_copy(bufs[b], out_hbm.at[t, pl.ds(b0, b_per_w)], ssems[b])

        def wait_store(t, b):
            pltpu.make_async_copy(
                bufs[b], out_hbm.at[t, pl.ds(b0, b_per_w)], ssems[b]
            ).wait()

        pltpu.sync_copy(
            x_hbm.at[pl.ds(0, PREFETCH), wid], idx_v.at[pl.ds(0, PREFETCH)]
        )
        tail_copy = pltpu.make_async_copy(
            x_hbm.at[pl.ds(PREFETCH, nstep - PREFETCH), wid],
            idx_v.at[pl.ds(PREFETCH, nstep - PREFETCH)],
            xsem,
        )
        tail_copy.start()

        for b in range(PREFETCH):
            start_gather(b, b)

        tail_copy.wait()

        @pl.loop(0, nstep, step=NBUF, unroll=False)
        def _group(g):
            for b in range(NBUF):
                t = g + b
                f = t + PREFETCH
                fb = (b + PREFETCH) % NBUF

                @pl.when(f - NBUF >= 0)
                def _():
                    wait_store(f - NBUF, fb)

                @pl.when(f < nstep)
                def _():
                    start_gather(f, fb)

                wait_gather(t, b)
                start_store(t, b)

        for t in range(nstep - NBUF + PREFETCH, nstep):
            wait_store(t, t % NBUF)

    return gather_kernel, nw, b_per_w


def kernel(x, weight):
    n_b, n_s = x.shape
    vocab, dim = weight.shape
    gather_kernel, nw, b_per_w = _build(n_b, n_s, vocab, dim)
    x_cols = x.T.astype(jnp.int32).reshape(n_s, nw, b_per_w)
    out = gather_kernel(x_cols, weight)
    return out.transpose(1, 0, 2)


# speedup vs baseline: 1.0031x; 1.0031x over previous
import functools

import jax
import jax.numpy as jnp
from jax import lax
from jax.experimental import pallas as pl
from jax.experimental.pallas import tpu as pltpu
from jax.experimental.pallas import tpu_sc as plsc

NBUF = 5
PREFETCH = 3


@functools.lru_cache(maxsize=None)
def _build(n_b: int, n_s: int, vocab: int, dim: int):
    info = plsc.get_sparse_core_info()
    nw = info.num_cores * info.num_subcores
    b_per_w = n_b // nw
    nstep = n_s
    assert n_b % nw == 0 and b_per_w <= 128 and nstep % NBUF == 0

    mesh = plsc.VectorSubcoreMesh(core_axis_name="c", subcore_axis_name="s")

    @functools.partial(
        pl.kernel,
        mesh=mesh,
        out_type=jax.ShapeDtypeStruct((n_s, n_b, dim), jnp.float32),
        scratch_types=(
            [pltpu.VMEM((nstep, b_per_w), jnp.int32)]
            + [pltpu.VMEM((b_per_w, dim), jnp.float32) for _ in range(NBUF)]
            + [pltpu.SemaphoreType.DMA for _ in range(2 * NBUF + 1)]
        ),
    )
    def gather_kernel(x_hbm, w_hbm, out_hbm, idx_v, *rest):
        bufs = rest[:NBUF]
        gsems = rest[NBUF : 2 * NBUF]
        ssems = rest[2 * NBUF : 3 * NBUF]
        xsem = rest[3 * NBUF]
        wid = lax.axis_index("s") * info.num_cores + lax.axis_index("c")
        b0 = wid * b_per_w

        def start_gather(t, b):
            pltpu.async_copy(w_hbm.at[idx_v.at[t]], bufs[b], gsems[b])

        def wait_gather(t, b):
            pltpu.make_async_copy(w_hbm.at[idx_v.at[t]], bufs[b], gsems[b]).wait()

        def start_store(t, b):
            pltpu.async_copy(bufs[b], out_hbm.at[t, pl.ds(b0, b_per_w)], ssems[b])

        def wait_store(t, b):
            pltpu.make_async_copy(
                bufs[b], out_hbm.at[t, pl.ds(b0, b_per_w)], ssems[b]
            ).wait()

        pltpu.sync_copy(
            x_hbm.at[pl.ds(0, PREFETCH), wid], idx_v.at[pl.ds(0, PREFETCH)]
        )
        tail_copy = pltpu.make_async_copy(
            x_hbm.at[pl.ds(PREFETCH, nstep - PREFETCH), wid],
            idx_v.at[pl.ds(PREFETCH, nstep - PREFETCH)],
            xsem,
        )
        tail_copy.start()

        for b in range(PREFETCH):
            start_gather(b, b)

        tail_copy.wait()

        @pl.loop(0, nstep, step=NBUF, unroll=False)
        def _group(g):
            for b in range(NBUF):
                t = g + b
                f = t + PREFETCH
                fb = (b + PREFETCH) % NBUF

                @pl.when(f - NBUF >= 0)
                def _():
                    wait_store(f - NBUF, fb)

                @pl.when(f < nstep)
                def _():
                    start_gather(f, fb)

                wait_gather(t, b)
                start_store(t, b)

        for t in range(nstep - NBUF + PREFETCH, nstep):
            wait_store(t, t % NBUF)

    return gather_kernel, nw, b_per_w


def kernel(x, weight):
    n_b, n_s = x.shape
    vocab, dim = weight.shape
    gather_kernel, nw, b_per_w = _build(n_b, n_s, vocab, dim)
    x_cols = x.T.astype(jnp.int32).reshape(n_s, nw, b_per_w)
    out = gather_kernel(x_cols, weight)
    return out.transpose(1, 0, 2)
